# Initial kernel scaffold; baseline (speedup 1.0000x reference)
import jax
import jax.numpy as jnp
from jax.experimental import pallas as pl


def kernel(x, edge_index, W1, b1, W2, b2, W3, b3, Wc, bc):
    raise NotImplementedError("write your pallas kernel here")


# speedup vs baseline: 78.1637x; 78.1637x over previous
import functools

import jax
import jax.numpy as jnp
from jax import lax
from jax.experimental import pallas as pl
from jax.experimental.pallas import tpu as pltpu
from jax.experimental.pallas import tpu_sc as plsc

N = 10000
E = 320000
NT = 16
N2 = 10240
NPT = N2 // NT
NCH = NPT // 16
EPT = E // NT
ECH = 4000
NECH = EPT // ECH

_B1, _B2, _B3, _BC, _W2, _W3, _WC = 0, 4, 8, 10, 11, 27, 35


def _rsqrt(v):
    i = plsc.bitcast(v, jnp.int32)
    i = jnp.int32(0x5F3759DF) - (i >> 1)
    y = plsc.bitcast(i, jnp.float32)
    for _ in range(3):
        y = y * (1.5 - 0.5 * v * y * y)
    return y


_EC = (0.9997394725061561, -0.9983509071103898, 0.49560486315569896,
       -0.1602332785811756, 0.036014005782361816, -0.005274476228013444,
       0.00037974512326439827)
_TC = (0.9999999471001855, -0.3333285614514503, 0.13326237128725457,
       -0.053561283847406535, 0.020694947793977743, -0.0069328785755570525,
       0.0016551237989875334, -0.00019545488075481374)


def _tanh(v):
    h = 0.5 * v
    u = h * h
    acc = _TC[7]
    for c in (_TC[6], _TC[5], _TC[4], _TC[3], _TC[2], _TC[1], _TC[0]):
        acc = acc * u + c
    t = acc * h
    d = 1.0 + t * t
    r = 1.0 / d
    r = r * (2.0 - d * r)
    t = (2.0 * t) * r
    y = jnp.minimum(jnp.abs(v), 8.0) * 0.25
    e = _EC[6]
    for c in (_EC[5], _EC[4], _EC[3], _EC[2], _EC[1], _EC[0]):
        e = e * y + c
    e = e * e
    e = e * e
    e = e * e
    te = 1.0 - 2.0 * e + 2.0 * e * e - 2.0 * e * e * e
    te = jnp.where(v < 0.0, -te, te)
    return jnp.where(jnp.abs(v) <= 2.5, t, te)


def _zero(ref):
    z = jnp.zeros((16,), jnp.float32)

    def zb(i, c):
        for q in range(8):
            ref[pl.ds(i * 128 + q * 16, 16)] = z
        return c

    lax.fori_loop(0, N2 // 128, zb, 0)


def _sc_body(h10_hbm, h11_hbm, h12_hbm, h13_hbm, es_hbm, ed_hbm, par_hbm,
             h_out_hbm, o_out_hbm,
             hc0, hc1, hc2, hc3, ac0, ac1, ac2, ac3, es0, ed0,
             st, dv, ew0, ew1, ew2, ew3, hbuf, obuf, pbuf,
             sem, semb, sem2, sal, sh0, sh1, sh2, sh3):
    hcs = [hc0, hc1, hc2, hc3]
    acs = [ac0, ac1, ac2, ac3]
    ews = [ew0, ew1, ew2, ew3]
    shs = [sh0, sh1, sh2, sh3]

    tid = lax.axis_index("s")
    nbase = tid * NPT
    ebase = tid * EPT
    is_last = tid == NT - 1

    pltpu.sync_copy(par_hbm, pbuf)
    pv = [pbuf[pl.ds(q * 16, 16)] for q in range(3)]

    def P(k):
        return pv[k // 16][k % 16]

    def stage_partials(sal, shadow_work=None):
        cp = pltpu.async_copy(sal.at[:, pl.ds(nbase, NPT)], st, sem2)
        if shadow_work is not None:
            shadow_work()
        cp.wait()

    def sum_partials(i):
        sl = pl.ds(i * 16, 16)
        v = st[0, sl]
        for t in range(1, NT):
            v = v + st[t, sl]
        return v

    import contextlib
    scope = jax.named_scope
    ones = jnp.ones((16,), jnp.float32)
    with scope("deg"):
      _zero(ac0)

      def dchunk(c, cr):
          pltpu.sync_copy(ed_hbm.at[pl.ds(ebase + c * ECH, ECH)], ed0)

          @plsc.parallel_loop(0, ECH // 80, unroll=2)
          def dloop(i):
            for u in range(5):
                d = ed0[pl.ds((i * 5 + u) * 16, 16)]
                plsc.addupdate_scatter(ac0, [d], ones)
          return cr

      lax.fori_loop(0, NECH, dchunk, 0)
      pltpu.sync_copy(ac0, sal.at[tid])
      plsc.subcore_barrier()

    def _zero123():
        for k in (1, 2, 3):
            _zero(acs[k])

    with scope("dinv"):
      stage_partials(sal, shadow_work=_zero123)

      def dinv_loop(i, cr):
        dv[pl.ds(i * 16, 16)] = _rsqrt(sum_partials(i) + 1.0)
        return cr

      lax.fori_loop(0, NCH, dinv_loop, 0)
      plsc.subcore_barrier()

    lane = jax.lax.iota(jnp.int32, 16)
    h1s = [h10_hbm, h11_hbm, h12_hbm, h13_hbm]
    with scope("h1"):
      h1cps = [pltpu.async_copy(h1s[c].at[pl.ds(nbase, NPT)], ews[c], sem2)
               for c in range(4)]
      for cp in h1cps:
        cp.wait()

      def h1_loop(i, cr):
        for q in range(2):
            sl = pl.ds((i * 2 + q) * 16, 16)
            d = dv[sl]
            for c in range(4):
                ews[c][sl] = d * ews[c][sl]
        return cr

      lax.fori_loop(0, NCH // 2, h1_loop, 0)
      for c in range(4):
        pltpu.sync_copy(ews[c], shs[c].at[pl.ds(nbase, NPT)])
      plsc.subcore_barrier()

    def agg_phase(ncols, zero_cols=()):
        hcps = [pltpu.async_copy(shs[k], hcs[k], sem2) for k in range(ncols)]
        sems = [sem, semb]
        for k in zero_cols:
            _zero(acs[k])
        for cp in hcps:
            cp.wait()

        def echunk(c, cr):
            pltpu.sync_copy(es_hbm.at[pl.ds(ebase + c * ECH, ECH)], es0)
            pltpu.sync_copy(ed_hbm.at[pl.ds(ebase + c * ECH, ECH)], ed0)

            @plsc.parallel_loop(0, ECH // 80, unroll=2)
            def eloop(i):
                for u in range(5):
                    sl = pl.ds((i * 5 + u) * 16, 16)
                    s = es0[sl]
                    d = ed0[sl]
                    for k in range(ncols):
                        v = plsc.load_gather(hcs[k], [s])
                        plsc.addupdate_scatter(acs[k], [d], v)
            return cr

        lax.fori_loop(0, NECH, echunk, 0)

    def ew_phase(ncols_in, ncols_out, b_off, w_off):
        for k in range(ncols_in):
            pltpu.sync_copy(acs[k], sal.at[tid])
            plsc.subcore_barrier()
            stage_partials(sal, shadow_work=lambda k=k: _zero(acs[k]))

            def tloop(i, cr):
                sl = pl.ds(i * 16, 16)
                agg = sum_partials(i) + hcs[k][pl.ds(nbase + i * 16, 16)]
                ews[k][sl] = _tanh(dv[sl] * agg + P(b_off + k))
                return cr

            lax.fori_loop(0, NCH, tloop, 0)
            plsc.subcore_barrier()

        def mloop(i, cr):
            sl = pl.ds(i * 16, 16)
            d = dv[sl]
            t = [ews[k][sl] for k in range(ncols_in)]
            h = []
            for j in range(ncols_out):
                acc = t[0] * P(w_off + j)
                for k in range(1, ncols_in):
                    acc = acc + t[k] * P(w_off + k * ncols_out + j)
                h.append(d * acc)
            for j in range(ncols_out):
                ews[j][sl] = h[j]
            return cr

        lax.fori_loop(0, NCH, mloop, 0)
        for j in range(ncols_out):
            pltpu.sync_copy(ews[j], shs[j].at[pl.ds(nbase, NPT)])
        plsc.subcore_barrier()

    with scope("agg1"):
        agg_phase(4, zero_cols=(0,))
    with scope("ew1"):
        ew_phase(4, 4, _B1, _W2)
    with scope("agg2"):
        agg_phase(4)
    with scope("ew2"):
        ew_phase(4, 2, _B2, _W3)
    with scope("agg3"):
        agg_phase(2)

    lane2 = lane * 2
    fin_scope = scope("fin")
    fin_scope.__enter__()
    for k in range(2):
        pltpu.sync_copy(acs[k], sal.at[tid])
        plsc.subcore_barrier()
        stage_partials(sal)

        def floop(i, cr):
            sl = pl.ds(i * 16, 16)
            agg = sum_partials(i) + hcs[k][pl.ds(nbase + i * 16, 16)]
            ews[k][sl] = _tanh(dv[sl] * agg + P(_B3 + k))
            return cr

        lax.fori_loop(0, NCH, floop, 0)
        plsc.subcore_barrier()

    def fin_loop(i, cr):
        for q in range(2):
            j2 = i * 2 + q
            sl = pl.ds(j2 * 16, 16)
            t0 = ews[0][sl]
            t1 = ews[1][sl]
            ii = lane2 + j2 * 32
            plsc.store_scatter(hbuf, [ii], t0)
            plsc.store_scatter(hbuf, [ii + 1], t1)
            obuf[sl] = t0 * P(_WC) + t1 * P(_WC + 1) + P(_BC)
        return cr

    lax.fori_loop(0, NCH // 2, fin_loop, 0)

    @pl.when(jnp.logical_not(is_last))
    def _():
        pltpu.sync_copy(hbuf.at[pl.ds(0, 1280)], h_out_hbm.at[pl.ds(2 * nbase, 1280)])
        pltpu.sync_copy(obuf.at[pl.ds(0, 640)], o_out_hbm.at[pl.ds(nbase, 640)])

    @pl.when(is_last)
    def _():
        pltpu.sync_copy(hbuf.at[pl.ds(0, 800)], h_out_hbm.at[pl.ds(19200, 800)])
        pltpu.sync_copy(obuf.at[pl.ds(0, 400)], o_out_hbm.at[pl.ds(9600, 400)])

    fin_scope.__exit__(None, None, None)


_sc_call = functools.partial(
    pl.kernel,
    out_type=(jax.ShapeDtypeStruct((2 * N,), jnp.float32),
              jax.ShapeDtypeStruct((N,), jnp.float32)),
    mesh=plsc.VectorSubcoreMesh(core_axis_name="c", subcore_axis_name="s",
                                num_cores=1),
    scratch_types=(
        [pltpu.VMEM((N2,), jnp.float32)] * 4
        + [pltpu.VMEM((N2,), jnp.float32)] * 4
        + [pltpu.VMEM((ECH,), jnp.int32)] * 2
        + [pltpu.VMEM((NT, NPT), jnp.float32)]
        + [pltpu.VMEM((NPT,), jnp.float32)] * 5
        + [pltpu.VMEM((1280,), jnp.float32)]
        + [pltpu.VMEM((640,), jnp.float32)]
        + [pltpu.VMEM((64,), jnp.float32)]
        + [pltpu.SemaphoreType.DMA] * 3
        + [pltpu.VMEM_SHARED((NT, N2), jnp.float32)]
        + [pltpu.VMEM_SHARED((N2,), jnp.float32)] * 4
    ),
    compiler_params=pltpu.CompilerParams(needs_layout_passes=False),
)(_sc_body)


def _mm_body(x_ref, w_ref, ei_ref, o_ref, es_ref, ed_ref):
    i = pl.program_id(0)
    h = jnp.dot(x_ref[...], w_ref[...], preferred_element_type=jnp.float32)
    rid = jax.lax.broadcasted_iota(jnp.int32, h.shape, 0) + i * 1024
    o_ref[...] = jnp.where(rid < N, h, 0.0).T
    es_ref[pl.ds(i * (E // 10), E // 10)] = ei_ref[0]
    ed_ref[pl.ds(i * (E // 10), E // 10)] = ei_ref[1]


_mm = pl.pallas_call(
    _mm_body,
    grid=(10,),
    in_specs=[pl.BlockSpec((1024, 128), lambda i: (i, 0)),
              pl.BlockSpec((128, 8), lambda i: (0, 0)),
              pl.BlockSpec((2, E // 10), lambda i: (0, i))],
    out_specs=[pl.BlockSpec((8, 1024), lambda i: (0, i)),
               pl.BlockSpec((E,), lambda i: (0,)),
               pl.BlockSpec((E,), lambda i: (0,))],
    out_shape=(jax.ShapeDtypeStruct((8, N2), jnp.float32),
               jax.ShapeDtypeStruct((E,), jnp.int32),
               jax.ShapeDtypeStruct((E,), jnp.int32)),
)


def kernel(x, edge_index, W1, b1, W2, b2, W3, b3, Wc, bc):
    w1p = jnp.pad(W1, ((0, 0), (0, 4)))
    h1t, es, ed = _mm(x, w1p, edge_index)
    h1c = [h1t[c] for c in range(4)]
    params = jnp.concatenate([
        b1, b2, b3, bc,
        W2.reshape(-1), W3.reshape(-1), Wc.reshape(-1),
        jnp.zeros((27,), jnp.float32),
    ])
    h_flat, o_flat = _sc_call(h1c[0], h1c[1], h1c[2], h1c[3], es, ed, params)
    return (o_flat.reshape(N, 1), h_flat.reshape(N, 2))


# speedup vs baseline: 83.7501x; 1.0715x over previous
import functools

import jax
import jax.numpy as jnp
from jax import lax
from jax.experimental import pallas as pl
from jax.experimental.pallas import tpu as pltpu
from jax.experimental.pallas import tpu_sc as plsc

N = 10000
E = 320000
NT = 16
N2 = 10240
NPT = N2 // NT
NCH = NPT // 16
EPT = E // NT
ECH = 4000
NECH = EPT // ECH

_B1, _B2, _B3, _BC, _W2, _W3, _WC = 0, 4, 8, 10, 11, 27, 35


def _rsqrt(v):
    i = plsc.bitcast(v, jnp.int32)
    i = jnp.int32(0x5F3759DF) - (i >> 1)
    y = plsc.bitcast(i, jnp.float32)
    for _ in range(3):
        y = y * (1.5 - 0.5 * v * y * y)
    return y


_EC = (0.9997394725061561, -0.9983509071103898, 0.49560486315569896,
       -0.1602332785811756, 0.036014005782361816, -0.005274476228013444,
       0.00037974512326439827)
_TC = (0.9999999471001855, -0.3333285614514503, 0.13326237128725457,
       -0.053561283847406535, 0.020694947793977743, -0.0069328785755570525,
       0.0016551237989875334, -0.00019545488075481374)


def _tanh(v):
    h = 0.5 * v
    u = h * h
    acc = _TC[7]
    for c in (_TC[6], _TC[5], _TC[4], _TC[3], _TC[2], _TC[1], _TC[0]):
        acc = acc * u + c
    t = acc * h
    d = 1.0 + t * t
    r = 1.0 / d
    r = r * (2.0 - d * r)
    t = (2.0 * t) * r
    y = jnp.minimum(jnp.abs(v), 8.0) * 0.25
    e = _EC[6]
    for c in (_EC[5], _EC[4], _EC[3], _EC[2], _EC[1], _EC[0]):
        e = e * y + c
    e = e * e
    e = e * e
    e = e * e
    te = 1.0 - 2.0 * e + 2.0 * e * e - 2.0 * e * e * e
    te = jnp.where(v < 0.0, -te, te)
    return jnp.where(jnp.abs(v) <= 2.5, t, te)


def _zero(ref):
    z = jnp.zeros((16,), jnp.float32)

    def zb(i, c):
        for q in range(8):
            ref[pl.ds(i * 128 + q * 16, 16)] = z
        return c

    lax.fori_loop(0, N2 // 128, zb, 0)


def _sc_body(h10_hbm, h11_hbm, h12_hbm, h13_hbm, es_hbm, ed_hbm, par_hbm,
             h_out_hbm, o_out_hbm,
             hc0, hc1, hc2, hc3, ac0, ac1, ac2, ac3, es0, ed0, es1, ed1,
             st, dv, ew0, ew1, ew2, ew3, hbuf, obuf, pbuf,
             sem, semb, sem2, sal, sh0, sh1, sh2, sh3):
    hcs = [hc0, hc1, hc2, hc3]
    acs = [ac0, ac1, ac2, ac3]
    ews = [ew0, ew1, ew2, ew3]
    shs = [sh0, sh1, sh2, sh3]

    tid = lax.axis_index("s")
    nbase = tid * NPT
    ebase = tid * EPT
    is_last = tid == NT - 1

    pltpu.sync_copy(par_hbm, pbuf)
    pv = [pbuf[pl.ds(q * 16, 16)] for q in range(3)]

    def P(k):
        return pv[k // 16][k % 16]

    def stage_partials(sal, shadow_work=None):
        cp = pltpu.async_copy(sal.at[:, pl.ds(nbase, NPT)], st, sem2)
        if shadow_work is not None:
            shadow_work()
        cp.wait()

    def sum_partials(i):
        sl = pl.ds(i * 16, 16)
        v = st[0, sl]
        for t in range(1, NT):
            v = v + st[t, sl]
        return v

    import contextlib
    scope = jax.named_scope
    ones = jnp.ones((16,), jnp.float32)
    with scope("deg"):
      _zero(ac0)

      def dpair(p, cr):
          c0 = p * 2
          cpa = pltpu.async_copy(ed_hbm.at[pl.ds(ebase + c0 * ECH, ECH)], ed0, sem)
          cpb = pltpu.async_copy(ed_hbm.at[pl.ds(ebase + (c0 + 1) * ECH, ECH)], ed1, semb)
          cpa.wait()

          @plsc.parallel_loop(0, ECH // 80, unroll=2)
          def dloop(i):
            for u in range(5):
                d = ed0[pl.ds((i * 5 + u) * 16, 16)]
                plsc.addupdate_scatter(ac0, [d], ones)
          cpb.wait()

          @plsc.parallel_loop(0, ECH // 80, unroll=2)
          def dloop2(i):
            for u in range(5):
                d = ed1[pl.ds((i * 5 + u) * 16, 16)]
                plsc.addupdate_scatter(ac0, [d], ones)
          return cr

      lax.fori_loop(0, NECH // 2, dpair, 0)
      pltpu.sync_copy(ed_hbm.at[pl.ds(ebase + 4 * ECH, ECH)], ed0)

      @plsc.parallel_loop(0, ECH // 80, unroll=2)
      def dtail(i):
        for u in range(5):
            d = ed0[pl.ds((i * 5 + u) * 16, 16)]
            plsc.addupdate_scatter(ac0, [d], ones)
      pltpu.sync_copy(ac0, sal.at[tid])
      plsc.subcore_barrier()

    def _zero123():
        for k in (1, 2, 3):
            _zero(acs[k])

    with scope("dinv"):
      stage_partials(sal, shadow_work=_zero123)

      def dinv_loop(i, cr):
        dv[pl.ds(i * 16, 16)] = _rsqrt(sum_partials(i) + 1.0)
        return cr

      lax.fori_loop(0, NCH, dinv_loop, 0)
      plsc.subcore_barrier()

    lane = jax.lax.iota(jnp.int32, 16)
    h1s = [h10_hbm, h11_hbm, h12_hbm, h13_hbm]
    with scope("h1"):
      h1cps = [pltpu.async_copy(h1s[c].at[pl.ds(nbase, NPT)], ews[c], sem2)
               for c in range(4)]
      for cp in h1cps:
        cp.wait()

      def h1_loop(i, cr):
        for q in range(2):
            sl = pl.ds((i * 2 + q) * 16, 16)
            d = dv[sl]
            for c in range(4):
                ews[c][sl] = d * ews[c][sl]
        return cr

      lax.fori_loop(0, NCH // 2, h1_loop, 0)
      for c in range(4):
        pltpu.sync_copy(ews[c], shs[c].at[pl.ds(nbase, NPT)])
      plsc.subcore_barrier()

    def agg_phase(ncols, zero_cols=()):
        hcps = [pltpu.async_copy(shs[k], hcs[k], sem2) for k in range(ncols)]
        sems = [sem, semb]
        for k in zero_cols:
            _zero(acs[k])
        for cp in hcps:
            cp.wait()

        def ebody(esb, edb):
            @plsc.parallel_loop(0, ECH // 80, unroll=2)
            def eloop(i):
                for u in range(5):
                    sl = pl.ds((i * 5 + u) * 16, 16)
                    s = esb[sl]
                    d = edb[sl]
                    for k in range(ncols):
                        v = plsc.load_gather(hcs[k], [s])
                        plsc.addupdate_scatter(acs[k], [d], v)

        def epair(p, cr):
            c0 = p * 2
            cpa = [pltpu.async_copy(es_hbm.at[pl.ds(ebase + c0 * ECH, ECH)], es0, sem),
                   pltpu.async_copy(ed_hbm.at[pl.ds(ebase + c0 * ECH, ECH)], ed0, sem)]
            cpb = [pltpu.async_copy(es_hbm.at[pl.ds(ebase + (c0 + 1) * ECH, ECH)], es1, semb),
                   pltpu.async_copy(ed_hbm.at[pl.ds(ebase + (c0 + 1) * ECH, ECH)], ed1, semb)]
            for cp in cpa:
                cp.wait()
            ebody(es0, ed0)
            for cp in cpb:
                cp.wait()
            ebody(es1, ed1)
            return cr

        lax.fori_loop(0, NECH // 2, epair, 0)
        pltpu.sync_copy(es_hbm.at[pl.ds(ebase + 4 * ECH, ECH)], es0)
        pltpu.sync_copy(ed_hbm.at[pl.ds(ebase + 4 * ECH, ECH)], ed0)
        ebody(es0, ed0)

    def ew_phase(ncols_in, ncols_out, b_off, w_off):
        for k in range(ncols_in):
            pltpu.sync_copy(acs[k], sal.at[tid])
            plsc.subcore_barrier()
            stage_partials(sal, shadow_work=lambda k=k: _zero(acs[k]))

            def tloop(i, cr):
                sl = pl.ds(i * 16, 16)
                agg = sum_partials(i) + hcs[k][pl.ds(nbase + i * 16, 16)]
                ews[k][sl] = _tanh(dv[sl] * agg + P(b_off + k))
                return cr

            lax.fori_loop(0, NCH, tloop, 0)
            plsc.subcore_barrier()

        def mloop(i, cr):
            sl = pl.ds(i * 16, 16)
            d = dv[sl]
            t = [ews[k][sl] for k in range(ncols_in)]
            h = []
            for j in range(ncols_out):
                acc = t[0] * P(w_off + j)
                for k in range(1, ncols_in):
                    acc = acc + t[k] * P(w_off + k * ncols_out + j)
                h.append(d * acc)
            for j in range(ncols_out):
                ews[j][sl] = h[j]
            return cr

        lax.fori_loop(0, NCH, mloop, 0)
        for j in range(ncols_out):
            pltpu.sync_copy(ews[j], shs[j].at[pl.ds(nbase, NPT)])
        plsc.subcore_barrier()

    with scope("agg1"):
        agg_phase(4, zero_cols=(0,))
    with scope("ew1"):
        ew_phase(4, 4, _B1, _W2)
    with scope("agg2"):
        agg_phase(4)
    with scope("ew2"):
        ew_phase(4, 2, _B2, _W3)
    with scope("agg3"):
        agg_phase(2)

    lane2 = lane * 2
    fin_scope = scope("fin")
    fin_scope.__enter__()
    for k in range(2):
        pltpu.sync_copy(acs[k], sal.at[tid])
        plsc.subcore_barrier()
        stage_partials(sal)

        def floop(i, cr):
            sl = pl.ds(i * 16, 16)
            agg = sum_partials(i) + hcs[k][pl.ds(nbase + i * 16, 16)]
            ews[k][sl] = _tanh(dv[sl] * agg + P(_B3 + k))
            return cr

        lax.fori_loop(0, NCH, floop, 0)
        plsc.subcore_barrier()

    def fin_loop(i, cr):
        for q in range(2):
            j2 = i * 2 + q
            sl = pl.ds(j2 * 16, 16)
            t0 = ews[0][sl]
            t1 = ews[1][sl]
            ii = lane2 + j2 * 32
            plsc.store_scatter(hbuf, [ii], t0)
            plsc.store_scatter(hbuf, [ii + 1], t1)
            obuf[sl] = t0 * P(_WC) + t1 * P(_WC + 1) + P(_BC)
        return cr

    lax.fori_loop(0, NCH // 2, fin_loop, 0)

    @pl.when(jnp.logical_not(is_last))
    def _():
        pltpu.sync_copy(hbuf.at[pl.ds(0, 1280)], h_out_hbm.at[pl.ds(2 * nbase, 1280)])
        pltpu.sync_copy(obuf.at[pl.ds(0, 640)], o_out_hbm.at[pl.ds(nbase, 640)])

    @pl.when(is_last)
    def _():
        pltpu.sync_copy(hbuf.at[pl.ds(0, 800)], h_out_hbm.at[pl.ds(19200, 800)])
        pltpu.sync_copy(obuf.at[pl.ds(0, 400)], o_out_hbm.at[pl.ds(9600, 400)])

    fin_scope.__exit__(None, None, None)


_sc_call = functools.partial(
    pl.kernel,
    out_type=(jax.ShapeDtypeStruct((2 * N,), jnp.float32),
              jax.ShapeDtypeStruct((N,), jnp.float32)),
    mesh=plsc.VectorSubcoreMesh(core_axis_name="c", subcore_axis_name="s",
                                num_cores=1),
    scratch_types=(
        [pltpu.VMEM((N2,), jnp.float32)] * 4
        + [pltpu.VMEM((N2,), jnp.float32)] * 4
        + [pltpu.VMEM((ECH,), jnp.int32)] * 4
        + [pltpu.VMEM((NT, NPT), jnp.float32)]
        + [pltpu.VMEM((NPT,), jnp.float32)] * 5
        + [pltpu.VMEM((1280,), jnp.float32)]
        + [pltpu.VMEM((640,), jnp.float32)]
        + [pltpu.VMEM((64,), jnp.float32)]
        + [pltpu.SemaphoreType.DMA] * 3
        + [pltpu.VMEM_SHARED((NT, N2), jnp.float32)]
        + [pltpu.VMEM_SHARED((N2,), jnp.float32)] * 4
    ),
    compiler_params=pltpu.CompilerParams(needs_layout_passes=False),
)(_sc_body)


def _mm_body(x_ref, w_ref, ei_ref, o_ref, es_ref, ed_ref):
    i = pl.program_id(0)
    h = jnp.dot(x_ref[...], w_ref[...], preferred_element_type=jnp.float32)
    rid = jax.lax.broadcasted_iota(jnp.int32, h.shape, 0) + i * 1024
    o_ref[...] = jnp.where(rid < N, h, 0.0).T
    es_ref[pl.ds(i * (E // 10), E // 10)] = ei_ref[0]
    ed_ref[pl.ds(i * (E // 10), E // 10)] = ei_ref[1]


_mm = pl.pallas_call(
    _mm_body,
    grid=(10,),
    in_specs=[pl.BlockSpec((1024, 128), lambda i: (i, 0)),
              pl.BlockSpec((128, 8), lambda i: (0, 0)),
              pl.BlockSpec((2, E // 10), lambda i: (0, i))],
    out_specs=[pl.BlockSpec((8, 1024), lambda i: (0, i)),
               pl.BlockSpec((E,), lambda i: (0,)),
               pl.BlockSpec((E,), lambda i: (0,))],
    out_shape=(jax.ShapeDtypeStruct((8, N2), jnp.float32),
               jax.ShapeDtypeStruct((E,), jnp.int32),
               jax.ShapeDtypeStruct((E,), jnp.int32)),
)


def kernel(x, edge_index, W1, b1, W2, b2, W3, b3, Wc, bc):
    w1p = jnp.pad(W1, ((0, 0), (0, 4)))
    h1t, es, ed = _mm(x, w1p, edge_index)
    h1c = [h1t[c] for c in range(4)]
    params = jnp.concatenate([
        b1, b2, b3, bc,
        W2.reshape(-1), W3.reshape(-1), Wc.reshape(-1),
        jnp.zeros((27,), jnp.float32),
    ])
    h_flat, o_flat = _sc_call(h1c[0], h1c[1], h1c[2], h1c[3], es, ed, params)
    return (o_flat.reshape(N, 1), h_flat.reshape(N, 2))
